# Initial kernel scaffold; baseline (speedup 1.0000x reference)
import jax
import jax.numpy as jnp
from jax.experimental import pallas as pl


def kernel(source_nodes, destination_nodes, negative_nodes, edge_times, edge_idxs, n_neighbors, neighbor_idx, neighbor_eidx, neighbor_ts, node_feats, edge_feats, w_t, b_t, Wq, Wk, Wv, W1, b1, W2, b2):
    raise NotImplementedError("write your pallas kernel here")


# speedup vs baseline: 5.0489x; 5.0489x over previous
import functools

import jax
import jax.numpy as jnp
from jax import lax
from jax.experimental import pallas as pl
from jax.experimental.pallas import tpu as pltpu
from jax.experimental.pallas import tpu_sc as plsc

_N_NODES = 10000
_N_EDGES = 320000
_D = 128
_DE = 16
_TD = 10
_B = 4096
_K = 20
_H = 2
_DH = 64
_NE = 3 * _B
_NR = _NE * _K

_NC = 2
_NS = 16
_NW = _NC * _NS
_ROWS_W = _NR // _NW
_CH = 480
_NCH = _ROWS_W // _CH
_SRC_W = _NE // _NW


def _gather_body(node_hbm, edge_hbm, nbidx_hbm, eidx_hbm, srcidx_hbm,
                 nb_out, e_out, src_out,
                 idx_v, eidx_v, rows_v, erows_v, sidx_v, srows_v,
                 sem0, sem1):
    wid = lax.axis_index("s") * _NC + lax.axis_index("c")

    sbase = wid * _SRC_W
    pltpu.sync_copy(srcidx_hbm.at[pl.ds(sbase, _SRC_W)], sidx_v)
    cps = pltpu.async_copy(node_hbm.at[sidx_v], srows_v, sem1)

    for c in range(_NCH):
        base = wid * _ROWS_W + c * _CH
        pltpu.sync_copy(nbidx_hbm.at[pl.ds(base, _CH)], idx_v)
        pltpu.sync_copy(eidx_hbm.at[pl.ds(base, _CH)], eidx_v)
        cp0 = pltpu.async_copy(node_hbm.at[idx_v], rows_v, sem0)
        cp1 = pltpu.async_copy(edge_hbm.at[eidx_v], erows_v, sem0)
        cp0.wait()
        cp1.wait()
        pltpu.sync_copy(rows_v, nb_out.at[pl.ds(base, _CH)])
        pltpu.sync_copy(erows_v, e_out.at[pl.ds(base, _CH)])

    cps.wait()
    pltpu.sync_copy(srows_v, src_out.at[pl.ds(sbase, _SRC_W)])


@functools.cache
def _make_gather():
    return pl.kernel(
        _gather_body,
        mesh=plsc.VectorSubcoreMesh(core_axis_name="c", subcore_axis_name="s"),
        compiler_params=pltpu.CompilerParams(use_tc_tiling_on_sc=False),
        out_type=[
        jax.ShapeDtypeStruct((_NR, _D), jnp.float32),
        jax.ShapeDtypeStruct((_NR, _DE), jnp.float32),
        jax.ShapeDtypeStruct((_NE, _D), jnp.float32),
    ],
    scratch_types=[
        pltpu.VMEM((_CH,), jnp.int32),
        pltpu.VMEM((_CH,), jnp.int32),
        pltpu.VMEM((_CH, _D), jnp.float32),
        pltpu.VMEM((_CH, _DE), jnp.float32),
        pltpu.VMEM((_SRC_W,), jnp.int32),
        pltpu.VMEM((_SRC_W, _D), jnp.float32),
        pltpu.SemaphoreType.DMA,
        pltpu.SemaphoreType.DMA,
    ],
    )


_BE = 256
_NBLK = _NE // _BE


def _dense_body(dt_ref, src_ref, nb_ref, e_ref, wt_ref, bt_ref,
                wq_ref, wkv_ref, w1_ref, b1_ref, w2_ref, b2_ref, out_ref):
    f32 = jnp.float32
    nb = nb_ref[...]
    ev = e_ref[...]
    te = jnp.cos(dt_ref[...] * wt_ref[...] + bt_ref[...])

    wkv = wkv_ref[...]
    kv = jnp.dot(nb, wkv[0:_D], preferred_element_type=f32)
    kv += jnp.dot(ev, wkv[_D:_D + _DE], preferred_element_type=f32)
    kv += jnp.dot(te, wkv[_D + _DE:], preferred_element_type=f32)

    src = src_ref[...]
    q = jnp.dot(src, wq_ref[0:_D], preferred_element_type=f32)
    q += jnp.dot(jnp.cos(bt_ref[...]), wq_ref[_D:], preferred_element_type=f32)

    k3 = kv[:, :_D].reshape(_BE, _K, _D)
    v3 = kv[:, _D:].reshape(_BE, _K, _D)
    scale = 1.0 / (_DH ** 0.5)

    w1 = w1_ref[...]
    hm = jnp.dot(src, w1[_H * _DH:], preferred_element_type=f32) + b1_ref[...]
    for h in range(_H):
        sl = slice(h * _DH, (h + 1) * _DH)
        qh = q[:, sl]
        kh = k3[:, :, sl]
        vh = v3[:, :, sl]
        logit = jnp.sum(kh * qh[:, None, :], axis=2) * scale
        m = jnp.max(logit, axis=1, keepdims=True)
        p = jnp.exp(logit - m)
        p = p / jnp.sum(p, axis=1, keepdims=True)
        oh = jnp.sum(vh * p[:, :, None], axis=1)
        hm += jnp.dot(oh, w1[h * _DH:(h + 1) * _DH], preferred_element_type=f32)

    hr = jnp.maximum(hm, 0.0)
    out_ref[...] = jnp.dot(hr, w2_ref[...], preferred_element_type=f32) + b2_ref[...]


def _dense(dt, src, nb, e, wt, bt, wq, wkv, w1, b1, w2, b2):
    bspec = lambda shp, imap: pl.BlockSpec(shp, imap)
    full = lambda shp: pl.BlockSpec(shp, lambda i: (0, 0))
    return pl.pallas_call(
        _dense_body,
        grid=(_NBLK,),
        in_specs=[
            bspec((_BE * _K, 1), lambda i: (i, 0)),
            bspec((_BE, _D), lambda i: (i, 0)),
            bspec((_BE * _K, _D), lambda i: (i, 0)),
            bspec((_BE * _K, _DE), lambda i: (i, 0)),
            full((1, _TD)),
            full((1, _TD)),
            full((_D + _TD, _D)),
            full((_D + _DE + _TD, 2 * _D)),
            full((_H * _DH + _D, _D)),
            full((1, _D)),
            full((_D, _D)),
            full((1, _D)),
        ],
        out_specs=bspec((_BE, _D), lambda i: (i, 0)),
        out_shape=jax.ShapeDtypeStruct((_NE, _D), jnp.float32),
    )(dt, src, nb, e, wt, bt, wq, wkv, w1, b1, w2, b2)


def kernel(source_nodes, destination_nodes, negative_nodes, edge_times,
           edge_idxs, n_neighbors, neighbor_idx, neighbor_eidx, neighbor_ts,
           node_feats, edge_feats, w_t, b_t, Wq, Wk, Wv, W1, b1, W2, b2):
    del edge_idxs, n_neighbors
    nodes = jnp.concatenate(
        [source_nodes, destination_nodes, negative_nodes]).astype(jnp.int32)
    nbidx = neighbor_idx.reshape(-1).astype(jnp.int32)
    eidx = neighbor_eidx.reshape(-1).astype(jnp.int32)
    ts3 = jnp.concatenate([edge_times, edge_times, edge_times])
    dt = (ts3[:, None] - neighbor_ts).reshape(_NR, 1)

    nb, e, src = _make_gather()(node_feats, edge_feats, nbidx, eidx, nodes)

    wkv = jnp.concatenate([Wk, Wv], axis=1)
    return _dense(dt, src, nb, e,
                  w_t.reshape(1, _TD), b_t.reshape(1, _TD),
                  Wq, wkv, W1, b1.reshape(1, _D), W2, b2.reshape(1, _D))


# speedup vs baseline: 5.0904x; 1.0082x over previous
import functools

import jax
import jax.numpy as jnp
from jax import lax
from jax.experimental import pallas as pl
from jax.experimental.pallas import tpu as pltpu
from jax.experimental.pallas import tpu_sc as plsc

_N_NODES = 10000
_N_EDGES = 320000
_D = 128
_DE = 16
_TD = 10
_B = 4096
_K = 20
_H = 2
_DH = 64
_NE = 3 * _B
_NR = _NE * _K

_NC = 2
_NS = 16
_NW = _NC * _NS
_ROWS_W = _NR // _NW
_CH = 480
_NCH = _ROWS_W // _CH
_SRC_W = _NE // _NW


def _gather_body(node_hbm, edge_hbm, nbidx_hbm, eidx_hbm, srcidx_hbm,
                 nb_out, e_out, src_out,
                 idx_v, eidx_v, rows_v, erows_v, sidx_v, srows_v,
                 sem0, sem1):
    wid = lax.axis_index("s") * _NC + lax.axis_index("c")

    sbase = wid * _SRC_W
    pltpu.sync_copy(srcidx_hbm.at[pl.ds(sbase, _SRC_W)], sidx_v)
    cps = pltpu.async_copy(node_hbm.at[sidx_v], srows_v, sem1)

    for c in range(_NCH):
        base = wid * _ROWS_W + c * _CH
        pltpu.sync_copy(nbidx_hbm.at[pl.ds(base, _CH)], idx_v)
        pltpu.sync_copy(eidx_hbm.at[pl.ds(base, _CH)], eidx_v)
        cp0 = pltpu.async_copy(node_hbm.at[idx_v], rows_v, sem0)
        cp1 = pltpu.async_copy(edge_hbm.at[eidx_v], erows_v, sem0)
        cp0.wait()
        cp1.wait()
        pltpu.sync_copy(rows_v, nb_out.at[pl.ds(base, _CH)])
        pltpu.sync_copy(erows_v, e_out.at[pl.ds(base, _CH)])

    cps.wait()
    pltpu.sync_copy(srows_v, src_out.at[pl.ds(sbase, _SRC_W)])


@functools.cache
def _make_gather():
    return pl.kernel(
        _gather_body,
        mesh=plsc.VectorSubcoreMesh(core_axis_name="c", subcore_axis_name="s"),
        compiler_params=pltpu.CompilerParams(use_tc_tiling_on_sc=False),
        out_type=[
        jax.ShapeDtypeStruct((_NR, _D), jnp.float32),
        jax.ShapeDtypeStruct((_NR, _DE), jnp.float32),
        jax.ShapeDtypeStruct((_NE, _D), jnp.float32),
    ],
    scratch_types=[
        pltpu.VMEM((_CH,), jnp.int32),
        pltpu.VMEM((_CH,), jnp.int32),
        pltpu.VMEM((_CH, _D), jnp.float32),
        pltpu.VMEM((_CH, _DE), jnp.float32),
        pltpu.VMEM((_SRC_W,), jnp.int32),
        pltpu.VMEM((_SRC_W, _D), jnp.float32),
        pltpu.SemaphoreType.DMA,
        pltpu.SemaphoreType.DMA,
    ],
    )


_BE = 256
_NBLK = _NE // _BE


def _dense_body(dt_ref, src_ref, nb_ref, e_ref, wt_ref, bt_ref,
                wq_ref, wkv_ref, w1_ref, b1_ref, w2_ref, b2_ref, out_ref):
    f32 = jnp.float32
    bf16 = jnp.bfloat16
    nb = nb_ref[...].astype(bf16)
    ev = e_ref[...].astype(bf16)
    te = jnp.cos(dt_ref[...] * wt_ref[...] + bt_ref[...])

    wkv = wkv_ref[...].astype(bf16)
    kv = jnp.dot(nb, wkv[0:_D], preferred_element_type=f32)
    kv += jnp.dot(ev, wkv[_D:_D + _DE], preferred_element_type=f32)
    kv += jnp.dot(te.astype(bf16), wkv[_D + _DE:], preferred_element_type=f32)

    src = src_ref[...]
    q = jnp.dot(src, wq_ref[0:_D], preferred_element_type=f32)
    q += jnp.dot(jnp.cos(bt_ref[...]), wq_ref[_D:], preferred_element_type=f32)

    k3 = kv[:, :_D].reshape(_BE, _K, _D)
    v3 = kv[:, _D:].reshape(_BE, _K, _D)
    scale = 1.0 / (_DH ** 0.5)

    w1 = w1_ref[...]
    hm = jnp.dot(src, w1[_H * _DH:], preferred_element_type=f32) + b1_ref[...]
    for h in range(_H):
        sl = slice(h * _DH, (h + 1) * _DH)
        qh = q[:, sl]
        kh = k3[:, :, sl]
        vh = v3[:, :, sl]
        logit = jnp.sum(kh * qh[:, None, :], axis=2) * scale
        m = jnp.max(logit, axis=1, keepdims=True)
        p = jnp.exp(logit - m)
        p = p / jnp.sum(p, axis=1, keepdims=True)
        oh = jnp.sum(vh * p[:, :, None], axis=1)
        hm += jnp.dot(oh, w1[h * _DH:(h + 1) * _DH], preferred_element_type=f32)

    hr = jnp.maximum(hm, 0.0)
    out_ref[...] = jnp.dot(hr, w2_ref[...], preferred_element_type=f32) + b2_ref[...]


def _dense(dt, src, nb, e, wt, bt, wq, wkv, w1, b1, w2, b2):
    bspec = lambda shp, imap: pl.BlockSpec(shp, imap)
    full = lambda shp: pl.BlockSpec(shp, lambda i: (0, 0))
    return pl.pallas_call(
        _dense_body,
        grid=(_NBLK,),
        in_specs=[
            bspec((_BE * _K, 1), lambda i: (i, 0)),
            bspec((_BE, _D), lambda i: (i, 0)),
            bspec((_BE * _K, _D), lambda i: (i, 0)),
            bspec((_BE * _K, _DE), lambda i: (i, 0)),
            full((1, _TD)),
            full((1, _TD)),
            full((_D + _TD, _D)),
            full((_D + _DE + _TD, 2 * _D)),
            full((_H * _DH + _D, _D)),
            full((1, _D)),
            full((_D, _D)),
            full((1, _D)),
        ],
        out_specs=bspec((_BE, _D), lambda i: (i, 0)),
        out_shape=jax.ShapeDtypeStruct((_NE, _D), jnp.float32),
    )(dt, src, nb, e, wt, bt, wq, wkv, w1, b1, w2, b2)


def kernel(source_nodes, destination_nodes, negative_nodes, edge_times,
           edge_idxs, n_neighbors, neighbor_idx, neighbor_eidx, neighbor_ts,
           node_feats, edge_feats, w_t, b_t, Wq, Wk, Wv, W1, b1, W2, b2):
    del edge_idxs, n_neighbors
    nodes = jnp.concatenate(
        [source_nodes, destination_nodes, negative_nodes]).astype(jnp.int32)
    nbidx = neighbor_idx.reshape(-1).astype(jnp.int32)
    eidx = neighbor_eidx.reshape(-1).astype(jnp.int32)
    ts3 = jnp.concatenate([edge_times, edge_times, edge_times])
    dt = (ts3[:, None] - neighbor_ts).reshape(_NR, 1)

    nb, e, src = _make_gather()(node_feats, edge_feats, nbidx, eidx, nodes)

    wkv = jnp.concatenate([Wk, Wv], axis=1)
    return _dense(dt, src, nb, e,
                  w_t.reshape(1, _TD), b_t.reshape(1, _TD),
                  Wq, wkv, W1, b1.reshape(1, _D), W2, b2.reshape(1, _D))


# speedup vs baseline: 6.9035x; 1.3562x over previous
import functools

import jax
import jax.numpy as jnp
from jax import lax
from jax.experimental import pallas as pl
from jax.experimental.pallas import tpu as pltpu
from jax.experimental.pallas import tpu_sc as plsc

_N_NODES = 10000
_N_EDGES = 320000
_D = 128
_DE = 16
_TD = 10
_B = 4096
_K = 20
_H = 2
_DH = 64
_NE = 3 * _B
_NR = _NE * _K

_NC = 2
_NS = 16
_NW = _NC * _NS
_ROWS_W = _NR // _NW
_CH = 480
_NCH = _ROWS_W // _CH
_SRC_W = _NE // _NW


def _gather_body(node_hbm, edge_hbm, nbidx_hbm, eidx_hbm, srcidx_hbm,
                 nb_out, e_out, src_out,
                 idx_v, eidx_v, rows_v, erows_v, sidx_v, srows_v,
                 sem0, sem1):
    wid = lax.axis_index("s") * _NC + lax.axis_index("c")

    sbase = wid * _SRC_W
    pltpu.sync_copy(srcidx_hbm.at[pl.ds(sbase, _SRC_W)], sidx_v)
    cps = pltpu.async_copy(node_hbm.at[sidx_v], srows_v, sem1)

    for c in range(_NCH):
        base = wid * _ROWS_W + c * _CH
        pltpu.sync_copy(nbidx_hbm.at[pl.ds(base, _CH)], idx_v)
        pltpu.sync_copy(eidx_hbm.at[pl.ds(base, _CH)], eidx_v)
        cp0 = pltpu.async_copy(node_hbm.at[idx_v], rows_v, sem0)
        cp1 = pltpu.async_copy(edge_hbm.at[eidx_v], erows_v, sem0)
        cp0.wait()
        cp1.wait()
        pltpu.sync_copy(rows_v, nb_out.at[pl.ds(base, _CH)])
        pltpu.sync_copy(erows_v, e_out.at[pl.ds(base, _CH)])

    cps.wait()
    pltpu.sync_copy(srows_v, src_out.at[pl.ds(sbase, _SRC_W)])


@functools.cache
def _make_gather():
    return pl.kernel(
        _gather_body,
        mesh=plsc.VectorSubcoreMesh(core_axis_name="c", subcore_axis_name="s"),
        compiler_params=pltpu.CompilerParams(use_tc_tiling_on_sc=False),
        out_type=[
        jax.ShapeDtypeStruct((_NR, _D), jnp.float32),
        jax.ShapeDtypeStruct((_NR, _DE), jnp.float32),
        jax.ShapeDtypeStruct((_NE, _D), jnp.float32),
    ],
    scratch_types=[
        pltpu.VMEM((_CH,), jnp.int32),
        pltpu.VMEM((_CH,), jnp.int32),
        pltpu.VMEM((_CH, _D), jnp.float32),
        pltpu.VMEM((_CH, _DE), jnp.float32),
        pltpu.VMEM((_SRC_W,), jnp.int32),
        pltpu.VMEM((_SRC_W, _D), jnp.float32),
        pltpu.SemaphoreType.DMA,
        pltpu.SemaphoreType.DMA,
    ],
    )


_BE = 256
_NBLK = _NE // _BE


def _dense_body(dt_ref, src_ref, nb_ref, e_ref, wt_ref, bt_ref,
                wq_ref, wkv_ref, w1_ref, b1_ref, w2_ref, b2_ref, bones_ref,
                out_ref):
    f32 = jnp.float32
    bf16 = jnp.bfloat16
    nb = nb_ref[...].reshape(_K * _BE, _D).astype(bf16)
    ev = e_ref[...].reshape(_K * _BE, _DE).astype(bf16)
    te = jnp.cos(dt_ref[...] * wt_ref[...] + bt_ref[...])

    wkv = wkv_ref[...].astype(bf16)
    kv = jnp.dot(nb, wkv[0:_D], preferred_element_type=f32)
    kv += jnp.dot(ev, wkv[_D:_D + _DE], preferred_element_type=f32)
    kv += jnp.dot(te.reshape(_K * _BE, _TD).astype(bf16), wkv[_D + _DE:],
                  preferred_element_type=f32)

    src = src_ref[...]
    q = jnp.dot(src, wq_ref[0:_D], preferred_element_type=f32)
    q += jnp.dot(jnp.cos(bt_ref[0]), wq_ref[_D:], preferred_element_type=f32)

    scale = 1.0 / (_DH ** 0.5)
    k3 = kv[:, :_D].reshape(_K, _BE, _D)
    v3 = kv[:, _D:].reshape(_K, _BE, _D)
    prod = k3 * q[None, :, :]
    logb = jnp.dot(prod.reshape(_K * _BE, _D), bones_ref[...],
                   preferred_element_type=f32).reshape(_K, _BE, _D) * scale
    mx = jnp.max(logb, axis=0, keepdims=True)
    p = jnp.exp(logb - mx)
    s = jnp.sum(p, axis=0)
    att = jnp.sum(p * v3, axis=0) / s

    w1 = w1_ref[...]
    hm = jnp.dot(src, w1[_H * _DH:], preferred_element_type=f32) + b1_ref[...]
    hm += jnp.dot(att, w1[:_H * _DH], preferred_element_type=f32)
    hr = jnp.maximum(hm, 0.0)
    out_ref[...] = jnp.dot(hr, w2_ref[...], preferred_element_type=f32) + b2_ref[...]


def _dense(dt, src, nb, e, wt, bt, wq, wkv, w1, b1, w2, b2, bones):
    b3 = lambda last: pl.BlockSpec((_K, _BE, last), lambda i: (0, i, 0))
    full = lambda shp: pl.BlockSpec(shp, lambda i: (0,) * len(shp))
    return pl.pallas_call(
        _dense_body,
        grid=(_NBLK,),
        in_specs=[
            b3(1),
            pl.BlockSpec((_BE, _D), lambda i: (i, 0)),
            b3(_D),
            b3(_DE),
            full((1, 1, _TD)),
            full((1, 1, _TD)),
            full((_D + _TD, _D)),
            full((_D + _DE + _TD, 2 * _D)),
            full((_H * _DH + _D, _D)),
            full((1, _D)),
            full((_D, _D)),
            full((1, _D)),
            full((_D, _D)),
        ],
        out_specs=pl.BlockSpec((_BE, _D), lambda i: (i, 0)),
        out_shape=jax.ShapeDtypeStruct((_NE, _D), jnp.float32),
    )(dt, src, nb, e, wt, bt, wq, wkv, w1, b1, w2, b2, bones)


def kernel(source_nodes, destination_nodes, negative_nodes, edge_times,
           edge_idxs, n_neighbors, neighbor_idx, neighbor_eidx, neighbor_ts,
           node_feats, edge_feats, w_t, b_t, Wq, Wk, Wv, W1, b1, W2, b2):
    del edge_idxs, n_neighbors
    nodes = jnp.concatenate(
        [source_nodes, destination_nodes, negative_nodes]).astype(jnp.int32)
    nbidx = neighbor_idx.astype(jnp.int32).T.reshape(-1)
    eidx = neighbor_eidx.astype(jnp.int32).T.reshape(-1)
    ts3 = jnp.concatenate([edge_times, edge_times, edge_times])
    dt = (ts3[:, None] - neighbor_ts).T.reshape(_K, _NE, 1)

    nb, e, src = _make_gather()(node_feats, edge_feats, nbidx, eidx, nodes)

    wkv = jnp.concatenate([Wk, Wv], axis=1)
    lane = jnp.arange(_D, dtype=jnp.int32)
    bones = (lane[:, None] // _DH == lane[None, :] // _DH).astype(jnp.float32)
    return _dense(dt, src, nb.reshape(_K, _NE, _D), e.reshape(_K, _NE, _DE),
                  w_t.reshape(1, 1, _TD), b_t.reshape(1, 1, _TD),
                  Wq, wkv, W1, b1.reshape(1, _D), W2, b2.reshape(1, _D), bones)


# speedup vs baseline: 11.5780x; 1.6771x over previous
import functools

import jax
import jax.numpy as jnp
import numpy as np
from jax import lax
from jax.experimental import pallas as pl
from jax.experimental.pallas import tpu as pltpu
from jax.experimental.pallas import tpu_sc as plsc

_N_NODES = 10000
_N_EDGES = 320000
_D = 128
_DE = 16
_TD = 10
_B = 4096
_K = 20
_H = 2
_DH = 64
_NE = 3 * _B
_NR = _NE * _K

_NC = 2
_NS = 16
_NW = _NC * _NS
_ROWS_W = _NR // _NW
_CH = 480
_NCH = _ROWS_W // _CH
_SRC_W = _NE // _NW


def _gather_body(node_hbm, edge_hbm, nbidx_hbm, eidx_hbm, srcidx_hbm,
                 nb_out, e_out, src_out,
                 idx_v, eidx_v, rows_v, erows_v, sidx_v, srows_v,
                 sem0, sem1):
    wid = lax.axis_index("s") * _NC + lax.axis_index("c")

    sbase = wid * _SRC_W
    pltpu.sync_copy(srcidx_hbm.at[pl.ds(sbase, _SRC_W)], sidx_v)
    cps = pltpu.async_copy(node_hbm.at[sidx_v], srows_v, sem1)

    for c in range(_NCH):
        base = wid * _ROWS_W + c * _CH
        pltpu.sync_copy(nbidx_hbm.at[pl.ds(base, _CH)], idx_v)
        pltpu.sync_copy(eidx_hbm.at[pl.ds(base, _CH)], eidx_v)
        cp0 = pltpu.async_copy(node_hbm.at[idx_v], rows_v, sem0)
        cp1 = pltpu.async_copy(edge_hbm.at[eidx_v], erows_v, sem0)
        cp0.wait()
        cp1.wait()
        pltpu.sync_copy(rows_v, nb_out.at[pl.ds(base, _CH)])
        pltpu.sync_copy(erows_v, e_out.at[pl.ds(base, _CH)])

    cps.wait()
    pltpu.sync_copy(srows_v, src_out.at[pl.ds(sbase, _SRC_W)])


@functools.cache
def _make_gather():
    return pl.kernel(
        _gather_body,
        mesh=plsc.VectorSubcoreMesh(core_axis_name="c", subcore_axis_name="s"),
        compiler_params=pltpu.CompilerParams(use_tc_tiling_on_sc=False),
        out_type=[
        jax.ShapeDtypeStruct((_NR, _D), jnp.float32),
        jax.ShapeDtypeStruct((_NR, _DE), jnp.float32),
        jax.ShapeDtypeStruct((_NE, _D), jnp.float32),
    ],
    scratch_types=[
        pltpu.VMEM((_CH,), jnp.int32),
        pltpu.VMEM((_CH,), jnp.int32),
        pltpu.VMEM((_CH, _D), jnp.float32),
        pltpu.VMEM((_CH, _DE), jnp.float32),
        pltpu.VMEM((_SRC_W,), jnp.int32),
        pltpu.VMEM((_SRC_W, _D), jnp.float32),
        pltpu.SemaphoreType.DMA,
        pltpu.SemaphoreType.DMA,
    ],
    )


_BE = 256
_NBLK = _NE // _BE

_COSC = (1.0, -19.739208, 64.93939, -85.45666, 60.24213, -26.404669,
         7.8001313, -1.4531124)
def _rnd(x):
    half = jnp.where(x >= 0, jnp.float32(0.5), jnp.float32(-0.5))
    return (x + half).astype(jnp.int32).astype(jnp.float32)


def _cospoly(r):
    t = r * r
    acc = jnp.float32(_COSC[-1])
    for c in _COSC[-2::-1]:
        acc = acc * t + jnp.float32(c)
    return acc


def _split12(x):
    return jax.lax.bitcast_convert_type(
        jax.lax.bitcast_convert_type(x, jnp.int32) & jnp.int32(-4096),
        jnp.float32)


def _dense_body(dt_ref, src_ref, nb_ref, e_ref, whi_ref, wlo_ref, bred_ref,
                bredq_ref, wq_ref, wkv_ref, w1_ref, b1_ref, w2_ref, b2_ref,
                bones_ref, out_ref):
    f32 = jnp.float32
    bf16 = jnp.bfloat16
    nb = nb_ref[...].reshape(_K * _BE, _D).astype(bf16)
    ev = e_ref[...].reshape(_K * _BE, _DE).astype(bf16)

    dt = dt_ref[...]
    dthi = _split12(dt)
    dtlo = dt - dthi
    whi = whi_ref[...]
    wlo = wlo_ref[...]
    u1 = dthi * whi
    r1 = u1 - _rnd(u1)
    u2 = dthi * wlo + dtlo * (whi + wlo)
    v = r1 + (u2 - _rnd(u2)) + bred_ref[...]
    v = v - _rnd(v)
    te3 = _cospoly(v).astype(bf16)

    wkv = wkv_ref[...].astype(bf16)
    kv = jnp.dot(nb, wkv[0:_D], preferred_element_type=f32)
    kv += jnp.dot(ev, wkv[_D:_D + _DE], preferred_element_type=f32)
    wt = wkv[_D + _DE:]
    lhsT = (((0,), (0,)), ((), ()))
    kv += jnp.concatenate(
        [jax.lax.dot_general(te3[k], wt, lhsT, preferred_element_type=f32)
         for k in range(_K)], axis=0)

    src = src_ref[...]
    q = jnp.dot(src, wq_ref[0:_D], preferred_element_type=f32)
    q += jnp.dot(_cospoly(bredq_ref[...]), wq_ref[_D:],
                 preferred_element_type=f32)

    scale = 1.0 / (_DH ** 0.5)
    k3 = kv[:, :_D].reshape(_K, _BE, _D)
    v3 = kv[:, _D:].reshape(_K, _BE, _D)
    prod = k3 * q[None, :, :]
    logb = jnp.dot(prod.reshape(_K * _BE, _D), bones_ref[...],
                   preferred_element_type=f32).reshape(_K, _BE, _D) * scale
    mx = jnp.max(logb, axis=0, keepdims=True)
    p = jnp.exp(logb - mx)
    s = jnp.sum(p, axis=0)
    att = jnp.sum(p * v3, axis=0) / s

    w1 = w1_ref[...]
    hm = jnp.dot(src, w1[_H * _DH:], preferred_element_type=f32) + b1_ref[...]
    hm += jnp.dot(att, w1[:_H * _DH], preferred_element_type=f32)
    hr = jnp.maximum(hm, 0.0)
    out_ref[...] = jnp.dot(hr, w2_ref[...], preferred_element_type=f32) + b2_ref[...]


def _dense(dt, src, nb, e, whi, wlo, bred, bredq, wq, wkv, w1, b1, w2, b2,
           bones):
    b3 = lambda last: pl.BlockSpec((_K, _BE, last), lambda i: (0, i, 0))
    full = lambda shp: pl.BlockSpec(shp, lambda i: (0,) * len(shp))
    return pl.pallas_call(
        _dense_body,
        grid=(_NBLK,),
        in_specs=[
            pl.BlockSpec((_K, 1, _BE), lambda i: (0, 0, i)),
            pl.BlockSpec((_BE, _D), lambda i: (i, 0)),
            b3(_D),
            b3(_DE),
            full((1, _TD, 1)),
            full((1, _TD, 1)),
            full((1, _TD, 1)),
            full((1, _TD)),
            full((_D + _TD, _D)),
            full((_D + _DE + _TD, 2 * _D)),
            full((_H * _DH + _D, _D)),
            full((1, _D)),
            full((_D, _D)),
            full((1, _D)),
            full((_D, _D)),
        ],
        out_specs=pl.BlockSpec((_BE, _D), lambda i: (i, 0)),
        out_shape=jax.ShapeDtypeStruct((_NE, _D), jnp.float32),
    )(dt, src, nb, e, whi, wlo, bred, bredq, wq, wkv, w1, b1, w2, b2, bones)


def kernel(source_nodes, destination_nodes, negative_nodes, edge_times,
           edge_idxs, n_neighbors, neighbor_idx, neighbor_eidx, neighbor_ts,
           node_feats, edge_feats, w_t, b_t, Wq, Wk, Wv, W1, b1, W2, b2):
    del edge_idxs, n_neighbors
    nodes = jnp.concatenate(
        [source_nodes, destination_nodes, negative_nodes]).astype(jnp.int32)
    nbidx = neighbor_idx.astype(jnp.int32).T.reshape(-1)
    eidx = neighbor_eidx.astype(jnp.int32).T.reshape(-1)
    ts3 = jnp.concatenate([edge_times, edge_times, edge_times])
    dt = (ts3[:, None] - neighbor_ts).T.reshape(_K, 1, _NE)

    nb, e, src = _make_gather()(node_feats, edge_feats, nbidx, eidx, nodes)

    inv2pi = 0.15915494309189535
    chi = float(np.float32(inv2pi))
    clo = float(np.float32(inv2pi - chi))
    chi_bits = np.float32(chi).view(np.int32)
    chh = float((chi_bits & np.int32(-4096)).view(np.float32))
    chl = chi - chh
    w = w_t.astype(jnp.float32)
    wh = _split12(w)
    wl = w - wh
    whi = _split12(w * chi)
    wlo = (wh * chh - whi) + (wh * chl + wl * chi) + w * clo
    bphase = b_t * chi + b_t * clo
    bred = bphase - jnp.round(bphase)

    wkv = jnp.concatenate([Wk, Wv], axis=1)
    lane = jnp.arange(_D, dtype=jnp.int32)
    bones = (lane[:, None] // _DH == lane[None, :] // _DH).astype(jnp.float32)
    return _dense(dt, src, nb.reshape(_K, _NE, _D), e.reshape(_K, _NE, _DE),
                  whi.reshape(1, _TD, 1), wlo.reshape(1, _TD, 1),
                  bred.reshape(1, _TD, 1), bred.reshape(1, _TD),
                  Wq, wkv, W1, b1.reshape(1, _D), W2, b2.reshape(1, _D), bones)


# speedup vs baseline: 14.9067x; 1.2875x over previous
import functools

import jax
import jax.numpy as jnp
import numpy as np
from jax import lax
from jax.experimental import pallas as pl
from jax.experimental.pallas import tpu as pltpu
from jax.experimental.pallas import tpu_sc as plsc

_N_NODES = 10000
_N_EDGES = 320000
_D = 128
_DE = 16
_TD = 10
_B = 4096
_K = 20
_H = 2
_DH = 64
_NE = 3 * _B
_NR = _NE * _K

_NC = 2
_NS = 16
_NW = _NC * _NS
_ROWS_W = _NR // _NW
_CH = 480
_NCH = _ROWS_W // _CH
_SRC_W = _NE // _NW


def _gather_nodes_body(node_hbm, nbidx_hbm, srcidx_hbm, nb_out, src_out,
                       idx_v, rows_v, sidx_v, srows_v, sem0, sem1):
    wid = lax.axis_index("s") * _NC + lax.axis_index("c")

    sbase = wid * _SRC_W
    pltpu.sync_copy(srcidx_hbm.at[pl.ds(sbase, _SRC_W)], sidx_v)
    cps = pltpu.async_copy(node_hbm.at[sidx_v], srows_v, sem1)

    for c in range(_NCH):
        base = wid * _ROWS_W + c * _CH
        pltpu.sync_copy(nbidx_hbm.at[pl.ds(base, _CH)], idx_v)
        pltpu.async_copy(node_hbm.at[idx_v], rows_v, sem0).wait()
        pltpu.sync_copy(rows_v, nb_out.at[pl.ds(base, _CH)])

    cps.wait()
    pltpu.sync_copy(srows_v, src_out.at[pl.ds(sbase, _SRC_W)])


_ECH = 1920
_ENCH = _ROWS_W // _ECH


def _gather_edges_body(edge_hbm, eidx_hbm, e_out, eidx_v, erows_v, sem0):
    wid = lax.axis_index("s") * _NC + lax.axis_index("c")
    for c in range(_ENCH):
        base = wid * _ROWS_W + c * _ECH
        pltpu.sync_copy(eidx_hbm.at[pl.ds(base, _ECH)], eidx_v)
        pltpu.async_copy(edge_hbm.at[eidx_v], erows_v, sem0).wait()
        pltpu.sync_copy(erows_v, e_out.at[pl.ds(base, _ECH), pl.ds(0, _DE)])


@functools.cache
def _make_gathers():
    gn = pl.kernel(
        _gather_nodes_body,
        mesh=plsc.VectorSubcoreMesh(core_axis_name="c", subcore_axis_name="s"),
        compiler_params=pltpu.CompilerParams(use_tc_tiling_on_sc=False),
        out_type=[
            jax.ShapeDtypeStruct((_NR, _D), jnp.float32),
            jax.ShapeDtypeStruct((_NE, _D), jnp.float32),
        ],
        scratch_types=[
            pltpu.VMEM((_CH,), jnp.int32),
            pltpu.VMEM((_CH, _D), jnp.float32),
            pltpu.VMEM((_SRC_W,), jnp.int32),
            pltpu.VMEM((_SRC_W, _D), jnp.float32),
            pltpu.SemaphoreType.DMA,
            pltpu.SemaphoreType.DMA,
        ],
    )
    ge = pl.kernel(
        _gather_edges_body,
        mesh=plsc.VectorSubcoreMesh(core_axis_name="c", subcore_axis_name="s"),
        compiler_params=pltpu.CompilerParams(use_tc_tiling_on_sc=False),
        out_type=[
            jax.ShapeDtypeStruct((_NR, _D), jnp.float32),
        ],
        scratch_types=[
            pltpu.VMEM((_ECH,), jnp.int32),
            pltpu.VMEM((_ECH, _DE), jnp.float32),
            pltpu.SemaphoreType.DMA,
        ],
    )
    return gn, ge


_BE = 256
_NBLK = _NE // _BE

_COSC = (1.0, -19.739208, 64.93939, -85.45666, 60.24213, -26.404669,
         7.8001313, -1.4531124)
def _rnd(x):
    half = jnp.where(x >= 0, jnp.float32(0.5), jnp.float32(-0.5))
    return (x + half).astype(jnp.int32).astype(jnp.float32)


def _cospoly(r):
    t = r * r
    acc = jnp.float32(_COSC[-1])
    for c in _COSC[-2::-1]:
        acc = acc * t + jnp.float32(c)
    return acc


def _split12(x):
    return jax.lax.bitcast_convert_type(
        jax.lax.bitcast_convert_type(x, jnp.int32) & jnp.int32(-4096),
        jnp.float32)


def _dense_body(dt_ref, src_ref, nb_ref, e_ref, whi_ref, wlo_ref, bred_ref,
                bredq_ref, wq_ref, wkv_ref, w1_ref, b1_ref, w2_ref, b2_ref,
                bones_ref, out_ref):
    f32 = jnp.float32
    bf16 = jnp.bfloat16
    nb = nb_ref[...].reshape(_K * _BE, _D).astype(bf16)
    ev = e_ref[:, :, 0:_DE].reshape(_K * _BE, _DE).astype(bf16)

    dt = dt_ref[...]
    dthi = _split12(dt)
    dtlo = dt - dthi
    whi = whi_ref[...]
    wlo = wlo_ref[...]
    u1 = dthi * whi
    r1 = u1 - _rnd(u1)
    u2 = dthi * wlo + dtlo * (whi + wlo)
    v = r1 + (u2 - _rnd(u2)) + bred_ref[...]
    v = v - _rnd(v)
    te3 = _cospoly(v).astype(bf16)

    wkv = wkv_ref[...].astype(bf16)
    kv = jnp.dot(nb, wkv[0:_D], preferred_element_type=f32)
    kv += jnp.dot(ev, wkv[_D:_D + _DE], preferred_element_type=f32)
    wt = wkv[_D + _DE:]
    lhsT = (((0,), (0,)), ((), ()))
    kv += jnp.concatenate(
        [jax.lax.dot_general(te3[k], wt, lhsT, preferred_element_type=f32)
         for k in range(_K)], axis=0)

    src = src_ref[...]
    q = jnp.dot(src, wq_ref[0:_D], preferred_element_type=f32)
    q += jnp.dot(_cospoly(bredq_ref[...]), wq_ref[_D:],
                 preferred_element_type=f32)

    scale = 1.0 / (_DH ** 0.5)
    k3 = kv[:, :_D].reshape(_K, _BE, _D)
    v3 = kv[:, _D:].reshape(_K, _BE, _D)
    prod = k3 * q[None, :, :]
    logb = jnp.dot(prod.reshape(_K * _BE, _D), bones_ref[...],
                   preferred_element_type=f32).reshape(_K, _BE, _D) * scale
    mx = jnp.max(logb, axis=0, keepdims=True)
    p = jnp.exp(logb - mx)
    s = jnp.sum(p, axis=0)
    att = jnp.sum(p * v3, axis=0) / s

    w1 = w1_ref[...]
    hm = jnp.dot(src, w1[_H * _DH:], preferred_element_type=f32) + b1_ref[...]
    hm += jnp.dot(att, w1[:_H * _DH], preferred_element_type=f32)
    hr = jnp.maximum(hm, 0.0)
    out_ref[...] = jnp.dot(hr, w2_ref[...], preferred_element_type=f32) + b2_ref[...]


def _dense(dt, src, nb, e, whi, wlo, bred, bredq, wq, wkv, w1, b1, w2, b2,
           bones):
    b3 = lambda last: pl.BlockSpec((_K, _BE, last), lambda i: (0, i, 0))
    full = lambda shp: pl.BlockSpec(shp, lambda i: (0,) * len(shp))
    return pl.pallas_call(
        _dense_body,
        grid=(_NBLK,),
        in_specs=[
            pl.BlockSpec((_K, 1, _BE), lambda i: (0, 0, i)),
            pl.BlockSpec((_BE, _D), lambda i: (i, 0)),
            b3(_D),
            b3(_D),
            full((1, _TD, 1)),
            full((1, _TD, 1)),
            full((1, _TD, 1)),
            full((1, _TD)),
            full((_D + _TD, _D)),
            full((_D + _DE + _TD, 2 * _D)),
            full((_H * _DH + _D, _D)),
            full((1, _D)),
            full((_D, _D)),
            full((1, _D)),
            full((_D, _D)),
        ],
        out_specs=pl.BlockSpec((_BE, _D), lambda i: (i, 0)),
        out_shape=jax.ShapeDtypeStruct((_NE, _D), jnp.float32),
    )(dt, src, nb, e, whi, wlo, bred, bredq, wq, wkv, w1, b1, w2, b2, bones)


def kernel(source_nodes, destination_nodes, negative_nodes, edge_times,
           edge_idxs, n_neighbors, neighbor_idx, neighbor_eidx, neighbor_ts,
           node_feats, edge_feats, w_t, b_t, Wq, Wk, Wv, W1, b1, W2, b2):
    del edge_idxs, n_neighbors
    nodes = jnp.concatenate(
        [source_nodes, destination_nodes, negative_nodes]).astype(jnp.int32)
    nbidx = neighbor_idx.astype(jnp.int32).T.reshape(-1)
    eidx = neighbor_eidx.astype(jnp.int32).T.reshape(-1)
    ts3 = jnp.concatenate([edge_times, edge_times, edge_times])
    dt = (ts3[:, None] - neighbor_ts).T.reshape(_K, 1, _NE)

    gn, ge = _make_gathers()
    nb, src = gn(node_feats, nbidx, nodes)
    (e,) = ge(edge_feats, eidx)

    inv2pi = 0.15915494309189535
    chi = float(np.float32(inv2pi))
    clo = float(np.float32(inv2pi - chi))
    chi_bits = np.float32(chi).view(np.int32)
    chh = float((chi_bits & np.int32(-4096)).view(np.float32))
    chl = chi - chh
    w = w_t.astype(jnp.float32)
    wh = _split12(w)
    wl = w - wh
    whi = _split12(w * chi)
    wlo = (wh * chh - whi) + (wh * chl + wl * chi) + w * clo
    bphase = b_t * chi + b_t * clo
    bred = bphase - jnp.round(bphase)

    wkv = jnp.concatenate([Wk, Wv], axis=1)
    lane = jnp.arange(_D, dtype=jnp.int32)
    bones = (lane[:, None] // _DH == lane[None, :] // _DH).astype(jnp.float32)
    return _dense(dt, src, nb.reshape(_K, _NE, _D), e.reshape(_K, _NE, _D),
                  whi.reshape(1, _TD, 1), wlo.reshape(1, _TD, 1),
                  bred.reshape(1, _TD, 1), bred.reshape(1, _TD),
                  Wq, wkv, W1, b1.reshape(1, _D), W2, b2.reshape(1, _D), bones)
